# baseline (speedup 1.0000x reference)
import functools

import jax
import jax.numpy as jnp
from jax import lax
from jax.experimental import pallas as pl
from jax.experimental.pallas import tpu as pltpu
from jax.experimental.pallas import tpu_sc as plsc

NC, NS, LANES = 2, 16, 16
NW = NC * NS
CHUNK = 128
DEG_W = 16


def _cdiv(a, b):
    return (a + b - 1) // b


def _mesh():
    return plsc.VectorSubcoreMesh(
        core_axis_name="c", subcore_axis_name="s",
        num_cores=NC, num_subcores=NS)


def _make_sc_degree(n_pad, cptot, dw=128):
    cpt = cptot // NW
    rpt = n_pad // NS
    kcol = dw // LANES

    def body(src_hbm, dst_hbm, out_hbm, idx_v, ones_v, acc_sh):
        cid = lax.axis_index("c")
        sid = lax.axis_index("s")
        wid = cid * NS + sid
        base = sid * rpt

        def zero_row(r, _):
            for k in range(kcol):
                ones_v[r, pl.ds(k * LANES, LANES)] = jnp.zeros(
                    (LANES,), jnp.float32)
            return 0
        lax.fori_loop(0, CHUNK, zero_row, 0)
        off = 0
        while off < rpt:
            m = min(CHUNK, rpt - off)
            pltpu.sync_copy(ones_v.at[pl.ds(0, m)],
                            acc_sh.at[pl.ds(base + off, m)])
            off += m
        plsc.subcore_barrier()

        for d in range(6):
            et, dr = divmod(d, 2)

            def fill_row(r, _):
                for k in range(kcol):
                    v = 1.0 if k == d else 0.0
                    ones_v[r, pl.ds(k * LANES, LANES)] = jnp.full(
                        (LANES,), v, jnp.float32)
                return 0
            lax.fori_loop(0, CHUNK, fill_row, 0)

            ih = src_hbm if dr == 0 else dst_hbm
            pltpu.sync_copy(ih.at[et, pl.ds(wid * cpt, cpt)], idx_v)

            def scat(j, _):
                pltpu.sync_copy(ones_v, acc_sh.at[idx_v.at[j]], add=True)
                return 0
            lax.fori_loop(0, cpt, scat, 0)

        plsc.subcore_barrier()
        pltpu.sync_copy(acc_sh.at[pl.ds(base, rpt)],
                        out_hbm.at[cid, pl.ds(base, rpt)])

    return pl.kernel(
        body,
        out_type=jax.ShapeDtypeStruct((NC, n_pad, dw), jnp.float32),
        mesh=_mesh(),
        scratch_types=[
            pltpu.VMEM((cpt, CHUNK), jnp.int32),
            pltpu.VMEM((CHUNK, dw), jnp.float32),
            pltpu.VMEM_SHARED((n_pad, dw), jnp.float32),
        ])


def _make_sc_agg(n_pad, dh, cptot):
    cpt = cptot // NW
    rpt = n_pad // NS
    rounds = cpt // 2
    kcol = dh // LANES

    def body(t0, t1, t2, src_hbm, dst_hbm, o0, o1, o2,
             idx_s, idx_d, rb0, rb1, acc_sh, g0, g1, s0, s1):
        cid = lax.axis_index("c")
        sid = lax.axis_index("s")
        wid = cid * NS + sid
        base = sid * rpt
        tbls = (t0, t1, t2)
        outs = (o0, o1, o2)

        for et in range(3):
            tbl = tbls[et]

            def zr(r, _):
                for k in range(kcol):
                    rb0[r, pl.ds(k * LANES, LANES)] = jnp.zeros(
                        (LANES,), jnp.float32)
                return 0
            lax.fori_loop(0, CHUNK, zr, 0)
            off = 0
            while off < rpt:
                m = min(CHUNK, rpt - off)
                pltpu.sync_copy(rb0.at[pl.ds(0, m)],
                                acc_sh.at[pl.ds(base + off, m)])
                off += m
            plsc.subcore_barrier()

            pltpu.sync_copy(src_hbm.at[et, pl.ds(wid * cpt, cpt)], idx_s)
            pltpu.sync_copy(dst_hbm.at[et, pl.ds(wid * cpt, cpt)], idx_d)

            pltpu.async_copy(tbl.at[idx_s.at[0]], rb0, g0)
            pltpu.async_copy(tbl.at[idx_s.at[1]], rb1, g1)

            def step(t, _):
                j0 = 2 * t
                j1 = 2 * t + 1
                pltpu.make_async_copy(tbl.at[idx_s.at[0]], rb0, g0).wait()
                pltpu.async_copy(rb0, acc_sh.at[idx_d.at[j0]], s0, add=True)
                pltpu.make_async_copy(tbl.at[idx_s.at[1]], rb1, g1).wait()
                pltpu.async_copy(rb1, acc_sh.at[idx_d.at[j1]], s1, add=True)
                pltpu.make_async_copy(rb0, acc_sh.at[idx_d.at[0]], s0).wait()
                pltpu.async_copy(
                    tbl.at[idx_s.at[lax.rem(j0 + 2, cpt)]], rb0, g0)
                pltpu.make_async_copy(rb1, acc_sh.at[idx_d.at[1]], s1).wait()
                pltpu.async_copy(
                    tbl.at[idx_s.at[lax.rem(j1 + 2, cpt)]], rb1, g1)
                return 0
            lax.fori_loop(0, rounds, step, 0)
            pltpu.make_async_copy(tbl.at[idx_s.at[0]], rb0, g0).wait()
            pltpu.make_async_copy(tbl.at[idx_s.at[1]], rb1, g1).wait()
            plsc.subcore_barrier()
            pltpu.sync_copy(acc_sh.at[pl.ds(base, rpt)],
                            outs[et].at[cid, pl.ds(base, rpt)])
            plsc.subcore_barrier()

    st = jax.ShapeDtypeStruct((NC, n_pad, dh), jnp.float32)
    return pl.kernel(
        body,
        out_type=(st, st, st),
        mesh=_mesh(),
        scratch_types=[
            pltpu.VMEM((cpt, CHUNK), jnp.int32),
            pltpu.VMEM((cpt, CHUNK), jnp.int32),
            pltpu.VMEM((CHUNK, dh), jnp.float32),
            pltpu.VMEM((CHUNK, dh), jnp.float32),
            pltpu.VMEM_SHARED((n_pad, dh), jnp.float32),
            pltpu.SemaphoreType.DMA,
            pltpu.SemaphoreType.DMA,
            pltpu.SemaphoreType.DMA,
            pltpu.SemaphoreType.DMA,
        ])


def _tc_deg_finalize(dp2, n):
    def body(d_ref, out_ref):
        v = d_ref[...]
        for d in range(6):
            s = v[d, :] + v[6 + d, :]
            out_ref[d, :] = lax.rsqrt(jnp.maximum(s[:n], 1.0))
        out_ref[6, :] = jnp.zeros((n,), jnp.float32)
        out_ref[7, :] = jnp.zeros((n,), jnp.float32)
    return pl.pallas_call(
        body, out_shape=jax.ShapeDtypeStruct((8, n), jnp.float32))(dp2)


def _tc_scale(h, dinv, n_pad):
    n, dh = h.shape

    def body(h_ref, di_ref, t0, t1, t2):
        hv = h_ref[...]
        for et, tr in enumerate((t0, t1, t2)):
            tr[:n, :] = hv * di_ref[:, 2 * et][:, None]
            tr[n:, :] = jnp.zeros((n_pad - n, dh), jnp.float32)
    st = jax.ShapeDtypeStruct((n_pad, dh), jnp.float32)
    return pl.pallas_call(body, out_shape=(st, st, st))(h, dinv)


def _tc_norm_scale(h_pre, stats, gamma, beta, dinv, n_pad):
    n, dh = h_pre.shape

    def body(h_ref, st_ref, g_ref, b_ref, di_ref, t0, t1, t2):
        mean = st_ref[0, :] / n
        var = st_ref[1, :] / n - mean * mean
        hn = ((h_ref[...] - mean[None, :]) * lax.rsqrt(var + 1e-5)[None, :]
              * g_ref[...] + b_ref[...])
        for et, tr in enumerate((t0, t1, t2)):
            tr[:n, :] = hn * di_ref[:, 2 * et][:, None]
            tr[n:, :] = jnp.zeros((n_pad - n, dh), jnp.float32)
    st = jax.ShapeDtypeStruct((n_pad, dh), jnp.float32)
    return pl.pallas_call(body, out_shape=(st, st, st))(
        h_pre, stats, gamma, beta, dinv)


def _tc_combine(p0, p1, p2, dinv, Wst, bt, fcW, fcb):
    n = dinv.shape[0]
    dh = fcW.shape[0]
    R = 1000
    grid = n // R

    def body(p0_ref, p1_ref, p2_ref, di_ref, w_ref, bt_ref, fcw_ref, fcb_ref,
             h_ref, st_ref):
        i = pl.program_id(0)
        acc = jnp.zeros((R, dh), jnp.float32)
        for et, pr in enumerate((p0_ref, p1_ref, p2_ref)):
            a = (pr[0] + pr[1]) * di_ref[:, 2 * et + 1][:, None]
            acc = acc + jnp.dot(a, w_ref[et],
                                preferred_element_type=jnp.float32)
        o1 = acc + bt_ref[...]
        h = jnp.dot(o1, fcw_ref[...], preferred_element_type=jnp.float32)
        h = jnp.maximum(h + fcb_ref[...], 0.0)
        h_ref[...] = h

        @pl.when(i == 0)
        def _():
            st_ref[...] = jnp.zeros((2, dh), jnp.float32)
        st_ref[...] += jnp.stack([jnp.sum(h, axis=0),
                                  jnp.sum(h * h, axis=0)])

    return pl.pallas_call(
        body,
        grid=(grid,),
        in_specs=[
            pl.BlockSpec((NC, R, dh), lambda i: (0, i, 0)),
            pl.BlockSpec((NC, R, dh), lambda i: (0, i, 0)),
            pl.BlockSpec((NC, R, dh), lambda i: (0, i, 0)),
            pl.BlockSpec((R, 8), lambda i: (i, 0)),
            pl.BlockSpec((3, dh, dh), lambda i: (0, 0, 0)),
            pl.BlockSpec((1, dh), lambda i: (0, 0)),
            pl.BlockSpec((dh, dh), lambda i: (0, 0)),
            pl.BlockSpec((1, dh), lambda i: (0, 0)),
        ],
        out_specs=[
            pl.BlockSpec((R, dh), lambda i: (i, 0)),
            pl.BlockSpec((2, dh), lambda i: (0, 0)),
        ],
        out_shape=[
            jax.ShapeDtypeStruct((n, dh), jnp.float32),
            jax.ShapeDtypeStruct((2, dh), jnp.float32),
        ],
    )(p0, p1, p2, dinv, Wst, bt, fcW, fcb)


def _tc_norm_final(h_pre, stats, gamma, beta):
    n, dh = h_pre.shape

    def body(h_ref, st_ref, g_ref, b_ref, out_ref):
        mean = st_ref[0, :] / n
        var = st_ref[1, :] / n - mean * mean
        out_ref[...] = ((h_ref[...] - mean[None, :])
                        * lax.rsqrt(var + 1e-5)[None, :]
                        * g_ref[...] + b_ref[...])
    return pl.pallas_call(
        body, out_shape=jax.ShapeDtypeStruct((n, dh), jnp.float32))(
            h_pre, stats, gamma, beta)


def kernel(x, params, edge_index_residue, edge_index_seq, edge_index_knn):
    n, _ = x.shape
    dh = params[0]['fcW'].shape[0]
    e = edge_index_residue.shape[1]
    n_pad = _cdiv(n + 1, NS * 8) * (NS * 8)
    e_pad = _cdiv(e, NW * CHUNK) * (NW * CHUNK)
    cptot = e_pad // CHUNK

    cpt = cptot // NW

    def prep(ei):
        pad = e_pad - e
        spread = n + (jnp.arange(pad, dtype=jnp.int32) % (n_pad - n))
        s = jnp.concatenate([ei[0].astype(jnp.int32), spread])
        d = jnp.concatenate([ei[1].astype(jnp.int32), spread])

        def shuf(a):
            return (a.reshape(cptot, CHUNK).reshape(cpt, NW, CHUNK)
                    .transpose(1, 0, 2).reshape(cptot, CHUNK))
        return shuf(s), shuf(d)

    s0, d0 = prep(edge_index_residue)
    s1, d1 = prep(edge_index_seq)
    s2, d2 = prep(edge_index_knn)
    src3 = jnp.stack([s0, s1, s2])
    dst3 = jnp.stack([d0, d1, d2])

    sc_degree = _make_sc_degree(n_pad, cptot)
    sc_agg = _make_sc_agg(n_pad, dh, cptot)

    dp = sc_degree(src3, dst3)
    dp2 = jnp.transpose(dp[:, :, 0:6 * DEG_W:DEG_W],
                        (0, 2, 1)).reshape(2 * 6, n_pad)
    dinv = _tc_deg_finalize(dp2, n).T

    h_pre = stats = gamma = beta = None
    for l, p in enumerate(params):
        Wst = jnp.stack([p['W_residue'], p['W_seq'], p['W_knn']])
        bt = (p['b_residue'] + p['b_seq'] + p['b_knn']).reshape(1, dh)
        if l == 0:
            t0, t1, t2 = _tc_scale(x, dinv, n_pad)
        else:
            t0, t1, t2 = _tc_norm_scale(h_pre, stats, gamma, beta, dinv, n_pad)
        p0, p1, p2 = sc_agg(t0, t1, t2, src3, dst3)
        h_pre, stats = _tc_combine(p0, p1, p2, dinv, Wst, bt,
                                   p['fcW'], p['fcb'].reshape(1, dh))
        gamma = p['gamma'].reshape(1, dh)
        beta = p['beta'].reshape(1, dh)
    return _tc_norm_final(h_pre, stats, gamma, beta)


# speedup vs baseline: 1.1522x; 1.1522x over previous
import functools

import jax
import jax.numpy as jnp
from jax import lax
from jax.experimental import pallas as pl
from jax.experimental.pallas import tpu as pltpu
from jax.experimental.pallas import tpu_sc as plsc

NC, NS, LANES = 2, 16, 16
NW = NC * NS
CHUNK = 128
DEG_W = 16


def _cdiv(a, b):
    return (a + b - 1) // b


def _mesh():
    return plsc.VectorSubcoreMesh(
        core_axis_name="c", subcore_axis_name="s",
        num_cores=NC, num_subcores=NS)


def _make_sc_degree(n_pad, cptot, dw=128):
    cpt = cptot // NW
    rpt = n_pad // NS
    kcol = dw // LANES

    def body(src_hbm, dst_hbm, out_hbm, idx_v, ones_v, acc_sh):
        cid = lax.axis_index("c")
        sid = lax.axis_index("s")
        wid = cid * NS + sid
        base = sid * rpt

        def zero_row(r, _):
            for k in range(kcol):
                ones_v[r, pl.ds(k * LANES, LANES)] = jnp.zeros(
                    (LANES,), jnp.float32)
            return 0
        lax.fori_loop(0, CHUNK, zero_row, 0)
        off = 0
        while off < rpt:
            m = min(CHUNK, rpt - off)
            pltpu.sync_copy(ones_v.at[pl.ds(0, m)],
                            acc_sh.at[pl.ds(base + off, m)])
            off += m
        plsc.subcore_barrier()

        for d in range(6):
            et, dr = divmod(d, 2)

            def fill_row(r, _):
                for k in range(kcol):
                    v = 1.0 if k == d else 0.0
                    ones_v[r, pl.ds(k * LANES, LANES)] = jnp.full(
                        (LANES,), v, jnp.float32)
                return 0
            lax.fori_loop(0, CHUNK, fill_row, 0)

            ih = src_hbm if dr == 0 else dst_hbm
            pltpu.sync_copy(ih.at[et, pl.ds(wid * cpt, cpt)], idx_v)

            def scat(j, _):
                pltpu.sync_copy(ones_v, acc_sh.at[idx_v.at[j]], add=True)
                return 0
            lax.fori_loop(0, cpt, scat, 0)

        plsc.subcore_barrier()
        pltpu.sync_copy(acc_sh.at[pl.ds(base, rpt)],
                        out_hbm.at[cid, pl.ds(base, rpt)])

    return pl.kernel(
        body,
        out_type=jax.ShapeDtypeStruct((NC, n_pad, dw), jnp.float32),
        mesh=_mesh(),
        scratch_types=[
            pltpu.VMEM((cpt, CHUNK), jnp.int32),
            pltpu.VMEM((CHUNK, dw), jnp.float32),
            pltpu.VMEM_SHARED((n_pad, dw), jnp.float32),
        ])


def _make_sc_agg(n_pad, dh, cptot):
    cpt = cptot // NW
    rpt = n_pad // NS
    rounds = cpt // 2
    kcol = dh // LANES

    def body(t0, t1, t2, src_hbm, dst_hbm, o0, o1, o2,
             idx_s, idx_d, rb0, rb1, acc_sh, g0, g1, s0, s1):
        cid = lax.axis_index("c")
        sid = lax.axis_index("s")
        wid = cid * NS + sid
        base = sid * rpt
        tbls = (t0, t1, t2)
        outs = (o0, o1, o2)

        for et in range(3):
            tbl = tbls[et]

            def zr(r, _):
                for k in range(kcol):
                    rb0[r, pl.ds(k * LANES, LANES)] = jnp.zeros(
                        (LANES,), jnp.float32)
                return 0
            lax.fori_loop(0, CHUNK, zr, 0)
            off = 0
            while off < rpt:
                m = min(CHUNK, rpt - off)
                pltpu.sync_copy(rb0.at[pl.ds(0, m)],
                                acc_sh.at[pl.ds(base + off, m)])
                off += m
            plsc.subcore_barrier()

            pltpu.sync_copy(src_hbm.at[et, pl.ds(wid * cpt, cpt)], idx_s)
            pltpu.sync_copy(dst_hbm.at[et, pl.ds(wid * cpt, cpt)], idx_d)

            pltpu.async_copy(tbl.at[idx_s.at[0]], rb0, g0)
            pltpu.async_copy(tbl.at[idx_s.at[1]], rb1, g1)

            def step(t, _):
                j0 = 2 * t
                j1 = 2 * t + 1
                pltpu.make_async_copy(tbl.at[idx_s.at[0]], rb0, g0).wait()
                pltpu.sync_copy(rb0, acc_sh.at[idx_d.at[j0]], add=True)
                pltpu.async_copy(
                    tbl.at[idx_s.at[lax.rem(j0 + 2, cpt)]], rb0, g0)
                pltpu.make_async_copy(tbl.at[idx_s.at[1]], rb1, g1).wait()
                pltpu.sync_copy(rb1, acc_sh.at[idx_d.at[j1]], add=True)
                pltpu.async_copy(
                    tbl.at[idx_s.at[lax.rem(j1 + 2, cpt)]], rb1, g1)
                return 0
            lax.fori_loop(0, rounds, step, 0)
            pltpu.make_async_copy(tbl.at[idx_s.at[0]], rb0, g0).wait()
            pltpu.make_async_copy(tbl.at[idx_s.at[1]], rb1, g1).wait()
            plsc.subcore_barrier()
            pltpu.sync_copy(acc_sh.at[pl.ds(base, rpt)],
                            outs[et].at[cid, pl.ds(base, rpt)])
            plsc.subcore_barrier()

    st = jax.ShapeDtypeStruct((NC, n_pad, dh), jnp.float32)
    return pl.kernel(
        body,
        out_type=(st, st, st),
        mesh=_mesh(),
        scratch_types=[
            pltpu.VMEM((cpt, CHUNK), jnp.int32),
            pltpu.VMEM((cpt, CHUNK), jnp.int32),
            pltpu.VMEM((CHUNK, dh), jnp.float32),
            pltpu.VMEM((CHUNK, dh), jnp.float32),
            pltpu.VMEM_SHARED((n_pad, dh), jnp.float32),
            pltpu.SemaphoreType.DMA,
            pltpu.SemaphoreType.DMA,
            pltpu.SemaphoreType.DMA,
            pltpu.SemaphoreType.DMA,
        ])


def _tc_deg_finalize(dp2, n):
    def body(d_ref, out_ref):
        v = d_ref[...]
        for d in range(6):
            s = v[d, :] + v[6 + d, :]
            out_ref[d, :] = lax.rsqrt(jnp.maximum(s[:n], 1.0))
        out_ref[6, :] = jnp.zeros((n,), jnp.float32)
        out_ref[7, :] = jnp.zeros((n,), jnp.float32)
    return pl.pallas_call(
        body, out_shape=jax.ShapeDtypeStruct((8, n), jnp.float32))(dp2)


def _tc_scale(h, dinv, n_pad):
    n, dh = h.shape

    def body(h_ref, di_ref, t0, t1, t2):
        hv = h_ref[...]
        for et, tr in enumerate((t0, t1, t2)):
            tr[:n, :] = hv * di_ref[:, 2 * et][:, None]
            tr[n:, :] = jnp.zeros((n_pad - n, dh), jnp.float32)
    st = jax.ShapeDtypeStruct((n_pad, dh), jnp.float32)
    return pl.pallas_call(body, out_shape=(st, st, st))(h, dinv)


def _tc_norm_scale(h_pre, stats, gamma, beta, dinv, n_pad):
    n, dh = h_pre.shape

    def body(h_ref, st_ref, g_ref, b_ref, di_ref, t0, t1, t2):
        mean = st_ref[0, :] / n
        var = st_ref[1, :] / n - mean * mean
        hn = ((h_ref[...] - mean[None, :]) * lax.rsqrt(var + 1e-5)[None, :]
              * g_ref[...] + b_ref[...])
        for et, tr in enumerate((t0, t1, t2)):
            tr[:n, :] = hn * di_ref[:, 2 * et][:, None]
            tr[n:, :] = jnp.zeros((n_pad - n, dh), jnp.float32)
    st = jax.ShapeDtypeStruct((n_pad, dh), jnp.float32)
    return pl.pallas_call(body, out_shape=(st, st, st))(
        h_pre, stats, gamma, beta, dinv)


def _tc_combine(p0, p1, p2, dinv, Wst, bt, fcW, fcb):
    n = dinv.shape[0]
    dh = fcW.shape[0]
    R = 1000
    grid = n // R

    def body(p0_ref, p1_ref, p2_ref, di_ref, w_ref, bt_ref, fcw_ref, fcb_ref,
             h_ref, st_ref):
        i = pl.program_id(0)
        acc = jnp.zeros((R, dh), jnp.float32)
        for et, pr in enumerate((p0_ref, p1_ref, p2_ref)):
            a = (pr[0] + pr[1]) * di_ref[:, 2 * et + 1][:, None]
            acc = acc + jnp.dot(a, w_ref[et],
                                preferred_element_type=jnp.float32)
        o1 = acc + bt_ref[...]
        h = jnp.dot(o1, fcw_ref[...], preferred_element_type=jnp.float32)
        h = jnp.maximum(h + fcb_ref[...], 0.0)
        h_ref[...] = h

        @pl.when(i == 0)
        def _():
            st_ref[...] = jnp.zeros((2, dh), jnp.float32)
        st_ref[...] += jnp.stack([jnp.sum(h, axis=0),
                                  jnp.sum(h * h, axis=0)])

    return pl.pallas_call(
        body,
        grid=(grid,),
        in_specs=[
            pl.BlockSpec((NC, R, dh), lambda i: (0, i, 0)),
            pl.BlockSpec((NC, R, dh), lambda i: (0, i, 0)),
            pl.BlockSpec((NC, R, dh), lambda i: (0, i, 0)),
            pl.BlockSpec((R, 8), lambda i: (i, 0)),
            pl.BlockSpec((3, dh, dh), lambda i: (0, 0, 0)),
            pl.BlockSpec((1, dh), lambda i: (0, 0)),
            pl.BlockSpec((dh, dh), lambda i: (0, 0)),
            pl.BlockSpec((1, dh), lambda i: (0, 0)),
        ],
        out_specs=[
            pl.BlockSpec((R, dh), lambda i: (i, 0)),
            pl.BlockSpec((2, dh), lambda i: (0, 0)),
        ],
        out_shape=[
            jax.ShapeDtypeStruct((n, dh), jnp.float32),
            jax.ShapeDtypeStruct((2, dh), jnp.float32),
        ],
    )(p0, p1, p2, dinv, Wst, bt, fcW, fcb)


def _tc_norm_final(h_pre, stats, gamma, beta):
    n, dh = h_pre.shape

    def body(h_ref, st_ref, g_ref, b_ref, out_ref):
        mean = st_ref[0, :] / n
        var = st_ref[1, :] / n - mean * mean
        out_ref[...] = ((h_ref[...] - mean[None, :])
                        * lax.rsqrt(var + 1e-5)[None, :]
                        * g_ref[...] + b_ref[...])
    return pl.pallas_call(
        body, out_shape=jax.ShapeDtypeStruct((n, dh), jnp.float32))(
            h_pre, stats, gamma, beta)


def kernel(x, params, edge_index_residue, edge_index_seq, edge_index_knn):
    n, _ = x.shape
    dh = params[0]['fcW'].shape[0]
    e = edge_index_residue.shape[1]
    n_pad = _cdiv(n + 1, NS * 8) * (NS * 8)
    e_pad = _cdiv(e, NW * CHUNK) * (NW * CHUNK)
    cptot = e_pad // CHUNK

    cpt = cptot // NW

    def prep(ei):
        pad = e_pad - e
        spread = n + (jnp.arange(pad, dtype=jnp.int32) % (n_pad - n))
        s = jnp.concatenate([ei[0].astype(jnp.int32), spread])
        d = jnp.concatenate([ei[1].astype(jnp.int32), spread])

        def shuf(a):
            return (a.reshape(cptot, CHUNK).reshape(cpt, NW, CHUNK)
                    .transpose(1, 0, 2).reshape(cptot, CHUNK))
        return shuf(s), shuf(d)

    s0, d0 = prep(edge_index_residue)
    s1, d1 = prep(edge_index_seq)
    s2, d2 = prep(edge_index_knn)
    src3 = jnp.stack([s0, s1, s2])
    dst3 = jnp.stack([d0, d1, d2])

    sc_degree = _make_sc_degree(n_pad, cptot)
    sc_agg = _make_sc_agg(n_pad, dh, cptot)

    dp = sc_degree(src3, dst3)
    dp2 = jnp.transpose(dp[:, :, 0:6 * DEG_W:DEG_W],
                        (0, 2, 1)).reshape(2 * 6, n_pad)
    dinv = _tc_deg_finalize(dp2, n).T

    h_pre = stats = gamma = beta = None
    for l, p in enumerate(params):
        Wst = jnp.stack([p['W_residue'], p['W_seq'], p['W_knn']])
        bt = (p['b_residue'] + p['b_seq'] + p['b_knn']).reshape(1, dh)
        if l == 0:
            t0, t1, t2 = _tc_scale(x, dinv, n_pad)
        else:
            t0, t1, t2 = _tc_norm_scale(h_pre, stats, gamma, beta, dinv, n_pad)
        p0, p1, p2 = sc_agg(t0, t1, t2, src3, dst3)
        h_pre, stats = _tc_combine(p0, p1, p2, dinv, Wst, bt,
                                   p['fcW'], p['fcb'].reshape(1, dh))
        gamma = p['gamma'].reshape(1, dh)
        beta = p['beta'].reshape(1, dh)
    return _tc_norm_final(h_pre, stats, gamma, beta)


# speedup vs baseline: 1.2926x; 1.1218x over previous
import functools

import jax
import jax.numpy as jnp
from jax import lax
from jax.experimental import pallas as pl
from jax.experimental.pallas import tpu as pltpu
from jax.experimental.pallas import tpu_sc as plsc

NC, NS, LANES = 2, 16, 16
NW = NC * NS
CHUNK = 128
DEG_W = 16


def _cdiv(a, b):
    return (a + b - 1) // b


def _mesh():
    return plsc.VectorSubcoreMesh(
        core_axis_name="c", subcore_axis_name="s",
        num_cores=NC, num_subcores=NS)


def _make_sc_degree(n_pad, cptot, dw=64):
    cpt = cptot // NW
    rpt = n_pad // NS
    kcol = dw // LANES
    bw = dw // 8

    def body(src_hbm, dst_hbm, out_hbm, idx_v, ones_v, acc_sh):
        cid = lax.axis_index("c")
        sid = lax.axis_index("s")
        wid = cid * NS + sid
        base = sid * rpt

        def zero_row(r, _):
            for k in range(kcol):
                ones_v[r, pl.ds(k * LANES, LANES)] = jnp.zeros(
                    (LANES,), jnp.float32)
            return 0
        lax.fori_loop(0, CHUNK, zero_row, 0)
        off = 0
        while off < rpt:
            m = min(CHUNK, rpt - off)
            pltpu.sync_copy(ones_v.at[pl.ds(0, m)],
                            acc_sh.at[pl.ds(base + off, m)])
            off += m
        plsc.subcore_barrier()

        for d in range(6):
            et, dr = divmod(d, 2)
            lo, hi = d * bw, (d + 1) * bw

            def fill_row(r, _):
                for k in range(kcol):
                    col = lax.iota(jnp.int32, LANES) + k * LANES
                    v = jnp.where((col >= lo) & (col < hi), 1.0, 0.0)
                    ones_v[r, pl.ds(k * LANES, LANES)] = v.astype(jnp.float32)
                return 0
            lax.fori_loop(0, CHUNK, fill_row, 0)

            ih = src_hbm if dr == 0 else dst_hbm
            pltpu.sync_copy(ih.at[et, pl.ds(wid * cpt, cpt)], idx_v)

            def scat(j, _):
                pltpu.sync_copy(ones_v, acc_sh.at[idx_v.at[j]], add=True)
                return 0
            lax.fori_loop(0, cpt, scat, 0)

        plsc.subcore_barrier()
        pltpu.sync_copy(acc_sh.at[pl.ds(base, rpt)],
                        out_hbm.at[cid, pl.ds(base, rpt)])

    return pl.kernel(
        body,
        out_type=jax.ShapeDtypeStruct((NC, n_pad, dw), jnp.float32),
        mesh=_mesh(),
        scratch_types=[
            pltpu.VMEM((cpt, CHUNK), jnp.int32),
            pltpu.VMEM((CHUNK, dw), jnp.float32),
            pltpu.VMEM_SHARED((n_pad, dw), jnp.float32),
        ])


def _make_sc_agg(n_pad, dh, cptot):
    cpt = cptot // NW
    rpt = n_pad // NS
    rounds = cpt // 2
    kcol = dh // LANES

    def body(t0, t1, t2, src_hbm, dst_hbm, o0, o1, o2,
             idx_s, idx_d, rb0, rb1, acc_sh, g0, g1, s0, s1):
        cid = lax.axis_index("c")
        sid = lax.axis_index("s")
        wid = cid * NS + sid
        base = sid * rpt
        tbls = (t0, t1, t2)
        outs = (o0, o1, o2)

        for et in range(3):
            tbl = tbls[et]

            def zr(r, _):
                for k in range(kcol):
                    rb0[r, pl.ds(k * LANES, LANES)] = jnp.zeros(
                        (LANES,), jnp.float32)
                return 0
            lax.fori_loop(0, CHUNK, zr, 0)
            off = 0
            while off < rpt:
                m = min(CHUNK, rpt - off)
                pltpu.sync_copy(rb0.at[pl.ds(0, m)],
                                acc_sh.at[pl.ds(base + off, m)])
                off += m
            plsc.subcore_barrier()

            pltpu.sync_copy(src_hbm.at[et, pl.ds(wid * cpt, cpt)], idx_s)
            pltpu.sync_copy(dst_hbm.at[et, pl.ds(wid * cpt, cpt)], idx_d)

            pltpu.async_copy(tbl.at[idx_s.at[0]], rb0, g0)
            pltpu.async_copy(tbl.at[idx_s.at[1]], rb1, g1)

            def step(t, _):
                j0 = 2 * t
                j1 = 2 * t + 1
                pltpu.make_async_copy(tbl.at[idx_s.at[0]], rb0, g0).wait()
                pltpu.sync_copy(rb0, acc_sh.at[idx_d.at[j0]], add=True)
                pltpu.async_copy(
                    tbl.at[idx_s.at[lax.rem(j0 + 2, cpt)]], rb0, g0)
                pltpu.make_async_copy(tbl.at[idx_s.at[1]], rb1, g1).wait()
                pltpu.sync_copy(rb1, acc_sh.at[idx_d.at[j1]], add=True)
                pltpu.async_copy(
                    tbl.at[idx_s.at[lax.rem(j1 + 2, cpt)]], rb1, g1)
                return 0
            lax.fori_loop(0, rounds, step, 0)
            pltpu.make_async_copy(tbl.at[idx_s.at[0]], rb0, g0).wait()
            pltpu.make_async_copy(tbl.at[idx_s.at[1]], rb1, g1).wait()
            plsc.subcore_barrier()
            pltpu.sync_copy(acc_sh.at[pl.ds(base, rpt)],
                            outs[et].at[cid, pl.ds(base, rpt)])
            plsc.subcore_barrier()

    st = jax.ShapeDtypeStruct((NC, n_pad, dh), jnp.float32)
    return pl.kernel(
        body,
        out_type=(st, st, st),
        mesh=_mesh(),
        scratch_types=[
            pltpu.VMEM((cpt, CHUNK), jnp.int32),
            pltpu.VMEM((cpt, CHUNK), jnp.int32),
            pltpu.VMEM((CHUNK, dh), jnp.float32),
            pltpu.VMEM((CHUNK, dh), jnp.float32),
            pltpu.VMEM_SHARED((n_pad, dh), jnp.float32),
            pltpu.SemaphoreType.DMA,
            pltpu.SemaphoreType.DMA,
            pltpu.SemaphoreType.DMA,
            pltpu.SemaphoreType.DMA,
        ])


def _tc_deg_finalize(dp2, n):
    def body(d_ref, out_ref):
        v = d_ref[...]
        for d in range(6):
            s = v[d, :] + v[6 + d, :]
            out_ref[d, :] = lax.rsqrt(jnp.maximum(s[:n], 1.0))
        out_ref[6, :] = jnp.zeros((n,), jnp.float32)
        out_ref[7, :] = jnp.zeros((n,), jnp.float32)
    return pl.pallas_call(
        body, out_shape=jax.ShapeDtypeStruct((8, n), jnp.float32))(dp2)


def _tc_scale(h, dinv, n_pad):
    n, dh = h.shape

    def body(h_ref, di_ref, t0, t1, t2):
        hv = h_ref[...]
        for et, tr in enumerate((t0, t1, t2)):
            tr[:n, :] = hv * di_ref[:, 2 * et][:, None]
            tr[n:, :] = jnp.zeros((n_pad - n, dh), jnp.float32)
    st = jax.ShapeDtypeStruct((n_pad, dh), jnp.float32)
    return pl.pallas_call(body, out_shape=(st, st, st))(h, dinv)


def _tc_norm_scale(h_pre, stats, gamma, beta, dinv, n_pad):
    n, dh = h_pre.shape

    def body(h_ref, st_ref, g_ref, b_ref, di_ref, t0, t1, t2):
        mean = st_ref[0, :] / n
        var = st_ref[1, :] / n - mean * mean
        hn = ((h_ref[...] - mean[None, :]) * lax.rsqrt(var + 1e-5)[None, :]
              * g_ref[...] + b_ref[...])
        for et, tr in enumerate((t0, t1, t2)):
            tr[:n, :] = hn * di_ref[:, 2 * et][:, None]
            tr[n:, :] = jnp.zeros((n_pad - n, dh), jnp.float32)
    st = jax.ShapeDtypeStruct((n_pad, dh), jnp.float32)
    return pl.pallas_call(body, out_shape=(st, st, st))(
        h_pre, stats, gamma, beta, dinv)


def _tc_combine(p0, p1, p2, dinv, Wst, bt, fcW, fcb):
    n = dinv.shape[0]
    dh = fcW.shape[0]
    R = 1000
    grid = n // R

    def body(p0_ref, p1_ref, p2_ref, di_ref, w_ref, bt_ref, fcw_ref, fcb_ref,
             h_ref, st_ref):
        i = pl.program_id(0)
        acc = jnp.zeros((R, dh), jnp.float32)
        for et, pr in enumerate((p0_ref, p1_ref, p2_ref)):
            a = (pr[0] + pr[1]) * di_ref[:, 2 * et + 1][:, None]
            acc = acc + jnp.dot(a, w_ref[et],
                                preferred_element_type=jnp.float32)
        o1 = acc + bt_ref[...]
        h = jnp.dot(o1, fcw_ref[...], preferred_element_type=jnp.float32)
        h = jnp.maximum(h + fcb_ref[...], 0.0)
        h_ref[...] = h

        @pl.when(i == 0)
        def _():
            st_ref[...] = jnp.zeros((2, dh), jnp.float32)
        st_ref[...] += jnp.stack([jnp.sum(h, axis=0),
                                  jnp.sum(h * h, axis=0)])

    return pl.pallas_call(
        body,
        grid=(grid,),
        in_specs=[
            pl.BlockSpec((NC, R, dh), lambda i: (0, i, 0)),
            pl.BlockSpec((NC, R, dh), lambda i: (0, i, 0)),
            pl.BlockSpec((NC, R, dh), lambda i: (0, i, 0)),
            pl.BlockSpec((R, 8), lambda i: (i, 0)),
            pl.BlockSpec((3, dh, dh), lambda i: (0, 0, 0)),
            pl.BlockSpec((1, dh), lambda i: (0, 0)),
            pl.BlockSpec((dh, dh), lambda i: (0, 0)),
            pl.BlockSpec((1, dh), lambda i: (0, 0)),
        ],
        out_specs=[
            pl.BlockSpec((R, dh), lambda i: (i, 0)),
            pl.BlockSpec((2, dh), lambda i: (0, 0)),
        ],
        out_shape=[
            jax.ShapeDtypeStruct((n, dh), jnp.float32),
            jax.ShapeDtypeStruct((2, dh), jnp.float32),
        ],
    )(p0, p1, p2, dinv, Wst, bt, fcW, fcb)


def _tc_norm_final(h_pre, stats, gamma, beta):
    n, dh = h_pre.shape

    def body(h_ref, st_ref, g_ref, b_ref, out_ref):
        mean = st_ref[0, :] / n
        var = st_ref[1, :] / n - mean * mean
        out_ref[...] = ((h_ref[...] - mean[None, :])
                        * lax.rsqrt(var + 1e-5)[None, :]
                        * g_ref[...] + b_ref[...])
    return pl.pallas_call(
        body, out_shape=jax.ShapeDtypeStruct((n, dh), jnp.float32))(
            h_pre, stats, gamma, beta)


def kernel(x, params, edge_index_residue, edge_index_seq, edge_index_knn):
    n, _ = x.shape
    dh = params[0]['fcW'].shape[0]
    e = edge_index_residue.shape[1]
    n_pad = _cdiv(n + 1, NS * 8) * (NS * 8)
    e_pad = _cdiv(e, NW * CHUNK) * (NW * CHUNK)
    cptot = e_pad // CHUNK

    cpt = cptot // NW

    def prep(ei):
        pad = e_pad - e
        spread = n + (jnp.arange(pad, dtype=jnp.int32) % (n_pad - n))
        s = jnp.concatenate([ei[0].astype(jnp.int32), spread])
        d = jnp.concatenate([ei[1].astype(jnp.int32), spread])

        def shuf(a):
            return (a.reshape(cptot, CHUNK).reshape(cpt, NW, CHUNK)
                    .transpose(1, 0, 2).reshape(cptot, CHUNK))
        return shuf(s), shuf(d)

    s0, d0 = prep(edge_index_residue)
    s1, d1 = prep(edge_index_seq)
    s2, d2 = prep(edge_index_knn)
    src3 = jnp.stack([s0, s1, s2])
    dst3 = jnp.stack([d0, d1, d2])

    sc_degree = _make_sc_degree(n_pad, cptot)
    sc_agg = _make_sc_agg(n_pad, dh, cptot)

    dp = sc_degree(src3, dst3)
    dp2 = jnp.transpose(dp[:, :, 0:6 * 8:8],
                        (0, 2, 1)).reshape(2 * 6, n_pad)
    dinv = _tc_deg_finalize(dp2, n).T

    h_pre = stats = gamma = beta = None
    for l, p in enumerate(params):
        Wst = jnp.stack([p['W_residue'], p['W_seq'], p['W_knn']])
        bt = (p['b_residue'] + p['b_seq'] + p['b_knn']).reshape(1, dh)
        if l == 0:
            t0, t1, t2 = _tc_scale(x, dinv, n_pad)
        else:
            t0, t1, t2 = _tc_norm_scale(h_pre, stats, gamma, beta, dinv, n_pad)
        p0, p1, p2 = sc_agg(t0, t1, t2, src3, dst3)
        h_pre, stats = _tc_combine(p0, p1, p2, dinv, Wst, bt,
                                   p['fcW'], p['fcb'].reshape(1, dh))
        gamma = p['gamma'].reshape(1, dh)
        beta = p['beta'].reshape(1, dh)
    return _tc_norm_final(h_pre, stats, gamma, beta)
